# Initial kernel scaffold; baseline (speedup 1.0000x reference)
import jax
import jax.numpy as jnp
from jax.experimental import pallas as pl


def kernel(x, knn_idx, knn_rel_pos, ln1_g, ln1_b, ln2_g, ln2_b, Wq, bq, Wk, bk, Wv, bv, Wo, bo, pa1_W, pa1_b, pa2_W, pa2_b, pv1_W, pv1_b, pv2_W, pv2_b, sm1_W, sm1_b, sm2_W, sm2_b, ffn1_W, ffn1_b, ffn2_W, ffn2_b):
    raise NotImplementedError("write your pallas kernel here")


# speedup vs baseline: 2.7312x; 2.7312x over previous
import functools

import jax
import jax.numpy as jnp
from jax import lax
from jax.experimental import pallas as pl
from jax.experimental.pallas import tpu as pltpu
import jax.experimental.pallas.tpu_sc as plsc

N = 10000
K = 32
DIM = 128
ATTN = 32
VAL = 32


_P1 = 2000


def _qkv_body(x_ref, wq_ref, bq_ref, wk_ref, bk_ref, wv_ref, bv_ref,
              q_ref, k_ref, v_ref):
    x = x_ref[...]
    mu = jnp.mean(x, axis=-1, keepdims=True)
    xc = x - mu
    var = jnp.mean(xc * xc, axis=-1, keepdims=True)
    h = xc * lax.rsqrt(var + 1e-5)
    q_ref[...] = jnp.dot(h, wq_ref[...], preferred_element_type=jnp.float32) + bq_ref[...]
    k_ref[...] = jnp.dot(h, wk_ref[...], preferred_element_type=jnp.float32) + bk_ref[...]
    v_ref[...] = jnp.dot(h, wv_ref[...], preferred_element_type=jnp.float32) + bv_ref[...]


def _qkv(x2d, wq, bq, wk, bk, wv, bv):
    grid = (N // _P1,)
    row_spec = pl.BlockSpec((_P1, DIM), lambda i: (i, 0))
    w_spec = pl.BlockSpec((DIM, ATTN), lambda i: (0, 0))
    b_spec = pl.BlockSpec((1, ATTN), lambda i: (0, 0))
    out_spec = pl.BlockSpec((_P1, ATTN), lambda i: (i, 0))
    return pl.pallas_call(
        _qkv_body,
        grid=grid,
        in_specs=[row_spec, w_spec, b_spec, w_spec, b_spec, w_spec, b_spec],
        out_specs=[out_spec, out_spec, out_spec],
        out_shape=[jax.ShapeDtypeStruct((N, ATTN), jnp.float32)] * 3,
    )(x2d, wq, bq, wk, bk, wv, bv)


_NC = 2
_NS = 16
_NW = _NC * _NS
_E_TOT = N * K
_B_PER_W = _E_TOT // _NW
_CHUNK = 1000
_NSTEPS = _B_PER_W // _CHUNK


def _gather_body(ktab, vtab, idx_hbm, kg_hbm, vg_hbm,
                 idx_v, kbuf, vbuf, sem_k, sem_v):
    wid = lax.axis_index("s") * _NC + lax.axis_index("c")
    base = wid * _B_PER_W

    def step(i, carry):
        off = base + i * _CHUNK
        pltpu.sync_copy(idx_hbm.at[pl.ds(off, _CHUNK)], idx_v)
        ck = pltpu.async_copy(ktab.at[idx_v], kbuf, sem_k)
        cv = pltpu.async_copy(vtab.at[idx_v], vbuf, sem_v)
        ck.wait()
        cv.wait()
        pltpu.sync_copy(kbuf, kg_hbm.at[pl.ds(off, _CHUNK)])
        pltpu.sync_copy(vbuf, vg_hbm.at[pl.ds(off, _CHUNK)])
        return carry

    lax.fori_loop(0, _NSTEPS, step, 0)


def _sc_gather(ktab, vtab, idx_flat):
    mesh = plsc.VectorSubcoreMesh(core_axis_name="c", subcore_axis_name="s",
                                  num_cores=_NC, num_subcores=_NS)
    fn = pl.kernel(
        _gather_body,
        out_type=[jax.ShapeDtypeStruct((_E_TOT, ATTN), jnp.float32),
                  jax.ShapeDtypeStruct((_E_TOT, VAL), jnp.float32)],
        mesh=mesh,
        compiler_params=pltpu.CompilerParams(use_tc_tiling_on_sc=False),
        scratch_types=[
            pltpu.VMEM((_CHUNK,), jnp.int32),
            pltpu.VMEM((_CHUNK, ATTN), jnp.float32),
            pltpu.VMEM((_CHUNK, VAL), jnp.float32),
            pltpu.SemaphoreType.DMA,
            pltpu.SemaphoreType.DMA,
        ],
    )
    return fn(ktab, vtab, idx_flat)


_P3 = 200
_E3 = _P3 * K


def _block_body(x_ref, q_ref, kg_ref, vg_ref, rp_ref,
                pa1w, pa1b, pa2w, pa2b, pv1w, pv1b, pv2w, pv2b,
                sm1w, sm1b, sm2r, sm2b,
                wo, bo, ln2g, ln2b, f1w, f1b, f2w, f2b,
                out_ref):
    rp = rp_ref[...]
    a1 = jnp.maximum(jnp.dot(rp, pa1w[...], preferred_element_type=jnp.float32) + pa1b[...], 0.0)
    ra = jnp.dot(a1, pa2w[...], preferred_element_type=jnp.float32) + pa2b[...]
    v1 = jnp.maximum(jnp.dot(rp, pv1w[...], preferred_element_type=jnp.float32) + pv1b[...], 0.0)
    rv = jnp.dot(v1, pv2w[...], preferred_element_type=jnp.float32) + pv2b[...]

    q = q_ref[...]
    qe = jnp.broadcast_to(q[:, None, :], (_P3, K, ATTN)).reshape(_E3, ATTN)
    t = jnp.tanh(qe - kg_ref[...] + ra)
    s1 = jnp.maximum(jnp.dot(t, sm1w[...], preferred_element_type=jnp.float32) + sm1b[...], 0.0)
    sc = jnp.dot(s1, sm2r[...], preferred_element_type=jnp.float32) + sm2b[...]
    sc3 = sc.reshape(_P3, K, ATTN)
    m = jnp.max(sc3, axis=1, keepdims=True)
    e = jnp.exp(sc3 - m)
    denom = jnp.sum(e, axis=1, keepdims=True)
    attn = e / denom

    vpr = (vg_ref[...] + rv).reshape(_P3, K, VAL)
    ctx = jnp.sum(attn * vpr, axis=1)

    x2 = x_ref[...] + jnp.dot(ctx, wo[...], preferred_element_type=jnp.float32) + bo[...]

    mu = jnp.mean(x2, axis=-1, keepdims=True)
    xc = x2 - mu
    var = jnp.mean(xc * xc, axis=-1, keepdims=True)
    h2 = xc * lax.rsqrt(var + 1e-5) * ln2g[...] + ln2b[...]

    f1 = jnp.dot(h2, f1w[...], preferred_element_type=jnp.float32) + f1b[...]
    g1 = f1 * 0.5 * (1.0 + lax.erf(f1 * (2.0 ** -0.5)))
    out_ref[...] = x2 + jnp.dot(g1, f2w[...], preferred_element_type=jnp.float32) + f2b[...]


def _attn_ffn(x2d, q_all, kg, vg, rp2, consts):
    grid = (N // _P3,)

    def fixed(shape):
        nd = len(shape)
        return pl.BlockSpec(shape, lambda i, _nd=nd: (0,) * _nd)

    in_specs = [
        pl.BlockSpec((_P3, DIM), lambda i: (i, 0)),
        pl.BlockSpec((_P3, ATTN), lambda i: (i, 0)),
        pl.BlockSpec((_E3, ATTN), lambda i: (i, 0)),
        pl.BlockSpec((_E3, VAL), lambda i: (i, 0)),
        pl.BlockSpec((_E3, 3), lambda i: (i, 0)),
    ] + [fixed(c.shape) for c in consts]
    return pl.pallas_call(
        _block_body,
        grid=grid,
        in_specs=in_specs,
        out_specs=pl.BlockSpec((_P3, DIM), lambda i: (i, 0)),
        out_shape=jax.ShapeDtypeStruct((N, DIM), jnp.float32),
    )(x2d, q_all, kg, vg, rp2, *consts)


def kernel(x, knn_idx, knn_rel_pos, ln1_g, ln1_b, ln2_g, ln2_b, Wq, bq, Wk, bk,
           Wv, bv, Wo, bo, pa1_W, pa1_b, pa2_W, pa2_b, pv1_W, pv1_b, pv2_W,
           pv2_b, sm1_W, sm1_b, sm2_W, sm2_b, ffn1_W, ffn1_b, ffn2_W, ffn2_b):
    x2d = x[0]
    wq = ln1_g[:, None] * Wq
    wk = ln1_g[:, None] * Wk
    wv = ln1_g[:, None] * Wv
    bq2 = (bq + ln1_b @ Wq)[None, :]
    bk2 = (bk + ln1_b @ Wk)[None, :]
    bv2 = (bv + ln1_b @ Wv)[None, :]
    q_all, k_all, v_all = _qkv(x2d, wq, bq2, wk, bk2, wv, bv2)

    idx_flat = knn_idx.reshape(-1).astype(jnp.int32)
    kg, vg = _sc_gather(k_all, v_all, idx_flat)

    rp2 = knn_rel_pos.reshape(_E_TOT, 3)
    inv = 1.0 / jnp.sqrt(jnp.float32(ATTN))
    sm2r = jnp.broadcast_to(sm2_W * inv, (ATTN, ATTN))
    sm2b = jnp.broadcast_to(sm2_b * inv, (1, ATTN))
    consts = [
        pa1_W, pa1_b[None, :], pa2_W, pa2_b[None, :],
        pv1_W, pv1_b[None, :], pv2_W, pv2_b[None, :],
        sm1_W, sm1_b[None, :], sm2r, sm2b,
        Wo, bo[None, :], ln2_g[None, :], ln2_b[None, :],
        ffn1_W, ffn1_b[None, :], ffn2_W, ffn2_b[None, :],
    ]
    out = _attn_ffn(x2d, q_all, kg, vg, rp2, consts)
    return out[None]


# speedup vs baseline: 3.8203x; 1.3988x over previous
import jax
import jax.numpy as jnp
from jax import lax
from jax.experimental import pallas as pl
from jax.experimental.pallas import tpu as pltpu
import jax.experimental.pallas.tpu_sc as plsc

N = 10000
K = 32
DIM = 128
ATTN = 32
VAL = 32


_P1 = 2000


def _qkv_body(x_ref, wq_ref, bq_ref, wkv_ref, bkv_ref, q_ref, kv_ref):
    x = x_ref[...]
    mu = jnp.mean(x, axis=-1, keepdims=True)
    xc = x - mu
    var = jnp.mean(xc * xc, axis=-1, keepdims=True)
    h = xc * lax.rsqrt(var + 1e-5)
    q_ref[...] = jnp.dot(h, wq_ref[...], preferred_element_type=jnp.float32) + bq_ref[...]
    kv_ref[...] = jnp.dot(h, wkv_ref[...], preferred_element_type=jnp.float32) + bkv_ref[...]


def _qkv(x2d, wq, bq, wkv, bkv):
    return pl.pallas_call(
        _qkv_body,
        grid=(N // _P1,),
        in_specs=[
            pl.BlockSpec((_P1, DIM), lambda i: (i, 0)),
            pl.BlockSpec((DIM, ATTN), lambda i: (0, 0)),
            pl.BlockSpec((1, ATTN), lambda i: (0, 0)),
            pl.BlockSpec((DIM, 2 * ATTN), lambda i: (0, 0)),
            pl.BlockSpec((1, 2 * ATTN), lambda i: (0, 0)),
        ],
        out_specs=[
            pl.BlockSpec((_P1, ATTN), lambda i: (i, 0)),
            pl.BlockSpec((_P1, 2 * ATTN), lambda i: (i, 0)),
        ],
        out_shape=[
            jax.ShapeDtypeStruct((N, ATTN), jnp.float32),
            jax.ShapeDtypeStruct((N, 2 * ATTN), jnp.float32),
        ],
    )(x2d, wq, bq, wkv, bkv)


_NC = 2
_NS = 16
_NW = _NC * _NS
_E_TOT = N * K
_B_PER_W = _E_TOT // _NW
_CHUNK = 1000
_NSTEPS = _B_PER_W // _CHUNK


def _gather_body(kvtab, idx_hbm, kvg_hbm, idx_v, buf, sem):
    wid = lax.axis_index("s") * _NC + lax.axis_index("c")
    base = wid * _B_PER_W

    def step(i, carry):
        off = base + i * _CHUNK
        pltpu.sync_copy(idx_hbm.at[pl.ds(off, _CHUNK)], idx_v)
        pltpu.async_copy(kvtab.at[idx_v], buf, sem).wait()
        pltpu.sync_copy(buf, kvg_hbm.at[pl.ds(off, _CHUNK)])
        return carry

    lax.fori_loop(0, _NSTEPS, step, 0)


def _sc_gather(kvtab, idx_flat):
    mesh = plsc.VectorSubcoreMesh(core_axis_name="c", subcore_axis_name="s",
                                  num_cores=_NC, num_subcores=_NS)
    fn = pl.kernel(
        _gather_body,
        out_type=jax.ShapeDtypeStruct((_E_TOT, 2 * ATTN), jnp.float32),
        mesh=mesh,
        compiler_params=pltpu.CompilerParams(use_tc_tiling_on_sc=False),
        scratch_types=[
            pltpu.VMEM((_CHUNK,), jnp.int32),
            pltpu.VMEM((_CHUNK, 2 * ATTN), jnp.float32),
            pltpu.SemaphoreType.DMA,
        ],
    )
    return fn(kvtab, idx_flat)


_P3 = 200
_R3 = _P3 * K // 2
_RPP = K // 2


def _block_body(x_ref, q_ref, kvg_ref, rp_ref,
                w1c, b1c, w2c, b2c, tq, wsm1, bsm1, msc, bsc, maskv,
                wo2, bo, ln2g, ln2b, f1w, f1b, f2w, f2b,
                out_ref):
    hid = jnp.maximum(jnp.dot(rp_ref[...], w1c[...], preferred_element_type=jnp.float32) + b1c[...], 0.0)
    rarv = jnp.dot(hid, w2c[...], preferred_element_type=jnp.float32) + b2c[...]

    qrow = jnp.dot(q_ref[...], tq[...], preferred_element_type=jnp.float32)
    qe = jnp.broadcast_to(qrow[:, None, :], (_P3, _RPP, DIM)).reshape(_R3, DIM)

    kvg = kvg_ref[...]
    u = jnp.tanh(qe - kvg + rarv)
    s1 = jnp.maximum(jnp.dot(u, wsm1[...], preferred_element_type=jnp.float32) + bsm1[...], 0.0)
    sc = jnp.dot(s1, msc[...], preferred_element_type=jnp.float32) + bsc[...]
    sc3 = sc.reshape(_P3, _RPP, DIM)
    mask = maskv[...]

    m = jnp.max(sc3, axis=1)
    mm = jnp.maximum(m, pltpu.roll(m, 64, axis=1))
    e = jnp.exp(sc3 - mm[:, None, :]) * mask[None]
    d = jnp.sum(e, axis=1)
    d2 = d + pltpu.roll(d, 64, axis=1) + (1.0 - mask)
    attn = e / d2[:, None, :]

    vpr = (kvg + rarv).reshape(_P3, _RPP, DIM)
    vsum = jnp.sum(attn * vpr, axis=1)

    x2 = x_ref[...] + jnp.dot(vsum, wo2[...], preferred_element_type=jnp.float32) + bo[...]

    mu = jnp.mean(x2, axis=-1, keepdims=True)
    xc = x2 - mu
    var = jnp.mean(xc * xc, axis=-1, keepdims=True)
    h2 = xc * lax.rsqrt(var + 1e-5) * ln2g[...] + ln2b[...]

    f1 = jnp.dot(h2, f1w[...], preferred_element_type=jnp.float32) + f1b[...]
    g1 = f1 * 0.5 * (1.0 + lax.erf(f1 * (2.0 ** -0.5)))
    out_ref[...] = x2 + jnp.dot(g1, f2w[...], preferred_element_type=jnp.float32) + f2b[...]


def _attn_ffn(x2d, q_all, kvg, rp6, consts):
    def fixed(shape):
        nd = len(shape)
        return pl.BlockSpec(shape, lambda i, _nd=nd: (0,) * _nd)

    in_specs = [
        pl.BlockSpec((_P3, DIM), lambda i: (i, 0)),
        pl.BlockSpec((_P3, ATTN), lambda i: (i, 0)),
        pl.BlockSpec((_R3, DIM), lambda i: (i, 0)),
        pl.BlockSpec((_R3, 6), lambda i: (i, 0)),
    ] + [fixed(c.shape) for c in consts]
    return pl.pallas_call(
        _block_body,
        grid=(N // _P3,),
        in_specs=in_specs,
        out_specs=pl.BlockSpec((_P3, DIM), lambda i: (i, 0)),
        out_shape=jax.ShapeDtypeStruct((N, DIM), jnp.float32),
    )(x2d, q_all, kvg, rp6, *consts)


def kernel(x, knn_idx, knn_rel_pos, ln1_g, ln1_b, ln2_g, ln2_b, Wq, bq, Wk, bk,
           Wv, bv, Wo, bo, pa1_W, pa1_b, pa2_W, pa2_b, pv1_W, pv1_b, pv2_W,
           pv2_b, sm1_W, sm1_b, sm2_W, sm2_b, ffn1_W, ffn1_b, ffn2_W, ffn2_b):
    f32 = jnp.float32
    x2d = x[0]
    wq = ln1_g[:, None] * Wq
    bq2 = (bq + ln1_b @ Wq)[None, :]
    wkv_raw = jnp.concatenate([Wk, Wv], axis=1)
    wkv = ln1_g[:, None] * wkv_raw
    bkv2 = (jnp.concatenate([bk, bv]) + ln1_b @ wkv_raw)[None, :]
    q_all, kv_all = _qkv(x2d, wq, bq2, wkv, bkv2)

    idx_flat = knn_idx.reshape(-1).astype(jnp.int32)
    kvg = _sc_gather(kv_all, idx_flat)
    kvg2 = kvg.reshape(_E_TOT // 2, DIM)
    rp6 = knn_rel_pos.reshape(_E_TOT // 2, 6)

    Z = jnp.zeros((ATTN, ATTN), f32)
    inv = 1.0 / jnp.sqrt(jnp.float32(ATTN))
    sm2r = jnp.broadcast_to(sm2_W * inv, (ATTN, ATTN))

    def four(b00, b01, b10, b11, b20, b21, b30, b31):
        top = jnp.concatenate([b00, b01, b10, b11], axis=1)
        bot = jnp.concatenate([b20, b21, b30, b31], axis=1)
        return top, bot

    w1c_top = jnp.concatenate([pa1_W, pv1_W, jnp.zeros((3, 2 * ATTN), f32)], axis=1)
    w1c_bot = jnp.concatenate([jnp.zeros((3, 2 * ATTN), f32), pa1_W, pv1_W], axis=1)
    w1c = jnp.concatenate([w1c_top, w1c_bot], axis=0)
    b1c = jnp.tile(jnp.concatenate([pa1_b, pv1_b]), 2)[None, :]

    r0, r1 = four(pa2_W, Z, Z, Z, Z, pv2_W, Z, Z)
    r2, r3 = four(Z, Z, pa2_W, Z, Z, Z, Z, pv2_W)
    w2c = jnp.concatenate([r0, r1, r2, r3], axis=0)
    b2c = jnp.tile(jnp.concatenate([pa2_b, pv2_b]), 2)[None, :]

    I = jnp.eye(ATTN, dtype=f32)
    tq = jnp.concatenate([I, Z, I, Z], axis=1)

    r0, r1 = four(sm1_W, Z, Z, Z, Z, Z, Z, Z)
    r2, r3 = four(Z, Z, sm1_W, Z, Z, Z, Z, Z)
    wsm1 = jnp.concatenate([r0, r1, r2, r3], axis=0)
    bsm1 = jnp.tile(jnp.concatenate([sm1_b, jnp.zeros((ATTN,), f32)]), 2)[None, :]

    r0, r1 = four(Z, sm2r, Z, Z, Z, Z, Z, Z)
    r2, r3 = four(Z, Z, Z, Z, Z, Z, Z, sm2r)
    msc = jnp.concatenate([r0, r1, r2, r3], axis=0)
    zb = jnp.zeros((ATTN,), f32)
    bsc = jnp.tile(jnp.concatenate([zb, jnp.full((ATTN,), sm2_b[0] * inv, f32)]), 2)[None, :]

    maskv = jnp.tile(jnp.concatenate([zb, jnp.ones((ATTN,), f32)]), 2)[None, :]
    wo2 = jnp.concatenate([jnp.zeros((ATTN, DIM), f32), Wo] * 2, axis=0)

    consts = [
        w1c, b1c, w2c, b2c, tq, wsm1, bsm1, msc, bsc, maskv,
        wo2, bo[None, :], ln2_g[None, :], ln2_b[None, :],
        ffn1_W, ffn1_b[None, :], ffn2_W, ffn2_b[None, :],
    ]
    out = _attn_ffn(x2d, q_all, kvg2, rp6, consts)
    return out[None]
